# baseline (speedup 1.0000x reference)
import functools

import jax
import jax.numpy as jnp
from jax import lax
from jax.experimental import pallas as pl
from jax.experimental.pallas import tpu as pltpu
from jax.experimental.pallas import tpu_sc as plsc

D = 128
NC = 2
NS = 16
NW = NC * NS
CHUNK = 128
LANES = 16
CORE0_FRAC = 1.0
SR = 80


def _mlp_block(h, w1_ref, b1_ref, w2_ref, b2_ref):
    h = jnp.dot(h, w1_ref[...], preferred_element_type=jnp.float32) + b1_ref[...]
    h = jnp.maximum(h, 0.0)
    return jnp.dot(h, w2_ref[...], preferred_element_type=jnp.float32) + b2_ref[...]


def _mlp1_body(x_ref, w1_ref, b1_ref, w2_ref, b2_ref, o_ref):
    o_ref[...] = _mlp_block(x_ref[...], w1_ref, b1_ref, w2_ref, b2_ref)


def _mlp2_body(x_ref, a_ref, w1_ref, b1_ref, w2_ref, b2_ref, o_ref):
    h = x_ref[...] + a_ref[0] + a_ref[1]
    o_ref[...] = _mlp_block(h, w1_ref, b1_ref, w2_ref, b2_ref)


_W_SPEC = pl.BlockSpec((D, D), lambda i: (0, 0))
_B_SPEC = pl.BlockSpec((1, D), lambda i: (0, 0))


def _mlp1(x, W1, b1, W2, b2, blk):
    n = x.shape[0]
    spec = pl.BlockSpec((blk, D), lambda i: (i, 0))
    return pl.pallas_call(
        _mlp1_body,
        grid=(n // blk,),
        in_specs=[spec, _W_SPEC, _B_SPEC, _W_SPEC, _B_SPEC],
        out_specs=spec,
        out_shape=jax.ShapeDtypeStruct((n, D), jnp.float32),
    )(x, W1, b1, W2, b2)


def _mlp2(x, parts, W1, b1, W2, b2, blk):
    n = x.shape[0]
    spec = pl.BlockSpec((blk, D), lambda i: (i, 0))
    part_spec = pl.BlockSpec((NC, blk, D), lambda i: (0, i, 0))
    return pl.pallas_call(
        _mlp2_body,
        grid=(n // blk,),
        in_specs=[spec, part_spec, _W_SPEC, _B_SPEC, _W_SPEC, _B_SPEC],
        out_specs=spec,
        out_shape=jax.ShapeDtypeStruct((n, D), jnp.float32),
    )(x, parts, W1, b1, W2, b2)


def _make_sc_scatter(n_pad, cpt0, cpt1):
    rows_per_tile = n_pad // NS
    wchunks = rows_per_tile // CHUNK
    wrem = rows_per_tile - wchunks * CHUNK
    mesh = plsc.VectorSubcoreMesh(core_axis_name="c", subcore_axis_name="s")

    rounds = (max(cpt0, cpt1) + SR - 1) // SR

    @functools.partial(
        pl.kernel,
        mesh=mesh,
        out_type=jax.ShapeDtypeStruct((NC, n_pad, D), jnp.float32),
        scratch_types=[
            pltpu.VMEM((SR, CHUNK), jnp.int32),
            pltpu.VMEM((2, CHUNK), jnp.int32),
            pltpu.VMEM((2, CHUNK), jnp.int32),
            pltpu.VMEM((CHUNK, D), jnp.float32),
            pltpu.VMEM((CHUNK, D), jnp.float32),
            pltpu.VMEM_SHARED((n_pad, D), jnp.float32),
            pltpu.SemaphoreType.DMA,
            pltpu.SemaphoreType.DMA,
            pltpu.SemaphoreType.DMA,
            pltpu.SemaphoreType.DMA,
        ],
    )
    def sc_kernel(h_hbm, pk_hbm, z_hbm, out_hbm,
                  pk_v, cidx, ridx, g0, g1, acc,
                  gsem0, gsem1, ssem0, ssem1):
        cid = lax.axis_index("c")
        sid = lax.axis_index("s")
        row_base = sid * rows_per_tile
        gbuf = (g0, g1)
        gsem = (gsem0, gsem1)
        ssem = (ssem0, ssem1)
        ebase = jnp.where(cid == 0, sid * cpt0, NS * cpt0 + sid * cpt1)
        my_cpt = jnp.where(cid == 0, cpt0, cpt1)

        with jax.named_scope("phase_init"):
            pltpu.sync_copy(z_hbm, g0)
            for k in range(wchunks):
                pltpu.sync_copy(g0, acc.at[pl.ds(row_base + k * CHUNK, CHUNK)])
            if wrem:
                pltpu.sync_copy(
                    g0.at[pl.ds(0, wrem)],
                    acc.at[pl.ds(row_base + wchunks * CHUNK, wrem)])
            plsc.subcore_barrier()

        def unpack(j, b):
            for c in range(CHUNK // LANES):
                v = pk_v[j, pl.ds(c * LANES, LANES)]
                cidx[b, pl.ds(c * LANES, LANES)] = lax.bitwise_and(v, 0xFFFF)
                ridx[b, pl.ds(c * LANES, LANES)] = lax.shift_right_logical(v, 16)

        def start_gather(b):
            pltpu.async_copy(h_hbm.at[cidx.at[b]], gbuf[b], gsem[b])

        def wait_gather(b):
            pltpu.make_async_copy(h_hbm.at[cidx.at[b]], gbuf[b], gsem[b]).wait()

        def start_scatter(b):
            pltpu.async_copy(gbuf[b], acc.at[ridx.at[b]], ssem[b], add=True)

        def wait_scatter(b):
            pltpu.make_async_copy(
                gbuf[b], acc.at[ridx.at[b]], ssem[b]).wait()

        with jax.named_scope("phase_main"):
            for r in range(rounds):
                npairs = jnp.clip(my_cpt - r * SR, 0, SR) // 2

                @pl.when(npairs > 0)
                def _(r=r, npairs=npairs):
                    pltpu.sync_copy(pk_hbm.at[pl.ds(ebase + r * SR, SR)], pk_v)
                    unpack(0, 0)
                    start_gather(0)

                    def pair(t, carry):
                        wait_gather(0)
                        start_scatter(0)

                        @pl.when(t > 0)
                        def _():
                            wait_scatter(1)
                        unpack(2 * t + 1, 1)
                        start_gather(1)

                        wait_gather(1)
                        start_scatter(1)
                        wait_scatter(0)

                        @pl.when(t < npairs - 1)
                        def _():
                            unpack(2 * t + 2, 0)
                            start_gather(0)
                        return carry
                    lax.fori_loop(0, npairs, pair, 0)
                    wait_scatter(1)

            plsc.subcore_barrier()

        with jax.named_scope("phase_writeout"):
            for k in range(wchunks):
                r0 = row_base + k * CHUNK
                pltpu.sync_copy(acc.at[pl.ds(r0, CHUNK)], g0)
                pltpu.sync_copy(g0, out_hbm.at[cid, pl.ds(r0, CHUNK)])
            if wrem:
                r0 = row_base + wchunks * CHUNK
                pltpu.sync_copy(acc.at[pl.ds(r0, wrem)], g0.at[pl.ds(0, wrem)])
                pltpu.sync_copy(g0.at[pl.ds(0, wrem)],
                                out_hbm.at[cid, pl.ds(r0, wrem)])

    return sc_kernel


def kernel(x, edge_index, W1, b1, W2, b2):
    n, d = x.shape
    e = edge_index.shape[1]
    assert d == D

    n_pad = ((n + 1 + NS * 8 - 1) // (NS * 8)) * (NS * 8)
    cpt_sum = (e + NS * CHUNK - 1) // (NS * CHUNK)
    cpt_sum = ((cpt_sum + 7) // 8) * 8
    cpt0 = min(max(int(round(cpt_sum * CORE0_FRAC / 8.0)) * 8, 0), cpt_sum)
    cpt1 = cpt_sum - cpt0
    rows = NS * cpt_sum + SR
    e_pad = rows * CHUNK

    row = edge_index[0].astype(jnp.int32)
    col = edge_index[1].astype(jnp.int32)
    pk = jnp.left_shift(row, 16) | col
    pk_p = jnp.concatenate(
        [pk, jnp.full((e_pad - e,), (n_pad - 1) << 16, jnp.int32)])
    pk2d = pk_p.reshape(rows, CHUNK)
    zeros = jnp.zeros((CHUNK, D), jnp.float32)

    b1r = b1.reshape(1, D)
    b2r = b2.reshape(1, D)

    h = _mlp1(x, W1, b1r, W2, b2r, blk=2000)
    parts = _make_sc_scatter(n_pad, cpt0, cpt1)(h, pk2d, zeros)
    return _mlp2(x, parts, W1, b1r, W2, b2r, blk=2000)


# speedup vs baseline: 1.2692x; 1.2692x over previous
import functools

import jax
import jax.numpy as jnp
from jax import lax
from jax.experimental import pallas as pl
from jax.experimental.pallas import tpu as pltpu
from jax.experimental.pallas import tpu_sc as plsc

D = 128
NC = 2
NS = 16
NW = NC * NS
CHUNK = 128
LANES = 16
CORE0_FRAC = 0.8
SR = 128


def _mlp_block(h, w1_ref, b1_ref, w2_ref, b2_ref):
    h = jnp.dot(h, w1_ref[...], preferred_element_type=jnp.float32) + b1_ref[...]
    h = jnp.maximum(h, 0.0)
    return jnp.dot(h, w2_ref[...], preferred_element_type=jnp.float32) + b2_ref[...]


def _mlp1_body(x_ref, w1_ref, b1_ref, w2_ref, b2_ref, o_ref):
    o_ref[...] = _mlp_block(x_ref[...], w1_ref, b1_ref, w2_ref, b2_ref)


def _mlp2_body(x_ref, a_ref, w1_ref, b1_ref, w2_ref, b2_ref, o_ref):
    h = x_ref[...] + a_ref[0] + a_ref[1]
    o_ref[...] = _mlp_block(h, w1_ref, b1_ref, w2_ref, b2_ref)


_W_SPEC = pl.BlockSpec((D, D), lambda i: (0, 0))
_B_SPEC = pl.BlockSpec((1, D), lambda i: (0, 0))


def _mlp1(x, W1, b1, W2, b2, blk):
    n = x.shape[0]
    spec = pl.BlockSpec((blk, D), lambda i: (i, 0))
    return pl.pallas_call(
        _mlp1_body,
        grid=(n // blk,),
        in_specs=[spec, _W_SPEC, _B_SPEC, _W_SPEC, _B_SPEC],
        out_specs=spec,
        out_shape=jax.ShapeDtypeStruct((n, D), jnp.float32),
    )(x, W1, b1, W2, b2)


def _mlp2(x, parts, W1, b1, W2, b2, blk):
    n = x.shape[0]
    spec = pl.BlockSpec((blk, D), lambda i: (i, 0))
    part_spec = pl.BlockSpec((NC, blk, D), lambda i: (0, i, 0))
    return pl.pallas_call(
        _mlp2_body,
        grid=(n // blk,),
        in_specs=[spec, part_spec, _W_SPEC, _B_SPEC, _W_SPEC, _B_SPEC],
        out_specs=spec,
        out_shape=jax.ShapeDtypeStruct((n, D), jnp.float32),
    )(x, parts, W1, b1, W2, b2)


def _make_sc_scatter(n_pad, cpt0, cpt1):
    rows_per_tile = n_pad // NS
    wchunks = rows_per_tile // CHUNK
    wrem = rows_per_tile - wchunks * CHUNK
    mesh = plsc.VectorSubcoreMesh(core_axis_name="c", subcore_axis_name="s")

    rounds = (max(cpt0, cpt1) + SR - 1) // SR

    @functools.partial(
        pl.kernel,
        mesh=mesh,
        out_type=jax.ShapeDtypeStruct((NC, n_pad, D), jnp.float32),
        scratch_types=[
            pltpu.VMEM((SR, CHUNK), jnp.int32),
            pltpu.VMEM((2, CHUNK), jnp.int32),
            pltpu.VMEM((2, CHUNK), jnp.int32),
            pltpu.VMEM((CHUNK, D), jnp.float32),
            pltpu.VMEM((CHUNK, D), jnp.float32),
            pltpu.VMEM_SHARED((n_pad, D), jnp.float32),
            pltpu.SemaphoreType.DMA,
            pltpu.SemaphoreType.DMA,
            pltpu.SemaphoreType.DMA,
            pltpu.SemaphoreType.DMA,
        ],
    )
    def sc_kernel(h_hbm, pk_hbm, z_hbm, out_hbm,
                  pk_v, cidx, ridx, g0, g1, acc,
                  gsem0, gsem1, ssem0, ssem1):
        cid = lax.axis_index("c")
        sid = lax.axis_index("s")
        row_base = sid * rows_per_tile
        gbuf = (g0, g1)
        gsem = (gsem0, gsem1)
        ssem = (ssem0, ssem1)
        ebase = jnp.where(cid == 0, sid * cpt0, NS * cpt0 + sid * cpt1)
        my_cpt = jnp.where(cid == 0, cpt0, cpt1)

        with jax.named_scope("phase_init"):
            pltpu.sync_copy(z_hbm, g0)
            for k in range(wchunks):
                pltpu.sync_copy(g0, acc.at[pl.ds(row_base + k * CHUNK, CHUNK)])
            if wrem:
                pltpu.sync_copy(
                    g0.at[pl.ds(0, wrem)],
                    acc.at[pl.ds(row_base + wchunks * CHUNK, wrem)])
            plsc.subcore_barrier()

        def unpack(j, b):
            for c in range(CHUNK // LANES):
                v = pk_v[j, pl.ds(c * LANES, LANES)]
                cidx[b, pl.ds(c * LANES, LANES)] = lax.bitwise_and(v, 0xFFFF)
                ridx[b, pl.ds(c * LANES, LANES)] = lax.shift_right_logical(v, 16)

        def start_gather(b):
            pltpu.async_copy(h_hbm.at[cidx.at[b]], gbuf[b], gsem[b])

        def wait_gather(b):
            pltpu.make_async_copy(h_hbm.at[cidx.at[b]], gbuf[b], gsem[b]).wait()

        def start_scatter(b):
            pltpu.async_copy(gbuf[b], acc.at[ridx.at[b]], ssem[b], add=True)

        def wait_scatter(b):
            pltpu.make_async_copy(
                gbuf[b], acc.at[ridx.at[b]], ssem[b]).wait()

        with jax.named_scope("phase_main"):
            for r in range(rounds):
                npairs = jnp.clip(my_cpt - r * SR, 0, SR) // 2

                @pl.when(npairs > 0)
                def _(r=r, npairs=npairs):
                    pltpu.sync_copy(pk_hbm.at[pl.ds(ebase + r * SR, SR)], pk_v)
                    unpack(0, 0)
                    start_gather(0)

                    def pair(t, carry):
                        wait_gather(0)
                        start_scatter(0)

                        @pl.when(t > 0)
                        def _():
                            wait_scatter(1)
                        unpack(2 * t + 1, 1)
                        start_gather(1)

                        wait_gather(1)
                        start_scatter(1)
                        wait_scatter(0)

                        @pl.when(t < npairs - 1)
                        def _():
                            unpack(2 * t + 2, 0)
                            start_gather(0)
                        return carry
                    lax.fori_loop(0, npairs, pair, 0)
                    wait_scatter(1)

            plsc.subcore_barrier()

        with jax.named_scope("phase_writeout"):
            for k in range(wchunks):
                r0 = row_base + k * CHUNK
                pltpu.sync_copy(acc.at[pl.ds(r0, CHUNK)], g0)
                pltpu.sync_copy(g0, out_hbm.at[cid, pl.ds(r0, CHUNK)])
            if wrem:
                r0 = row_base + wchunks * CHUNK
                pltpu.sync_copy(acc.at[pl.ds(r0, wrem)], g0.at[pl.ds(0, wrem)])
                pltpu.sync_copy(g0.at[pl.ds(0, wrem)],
                                out_hbm.at[cid, pl.ds(r0, wrem)])

    return sc_kernel


def kernel(x, edge_index, W1, b1, W2, b2):
    n, d = x.shape
    e = edge_index.shape[1]
    assert d == D

    n_pad = ((n + 1 + NS * 8 - 1) // (NS * 8)) * (NS * 8)
    cpt_sum = (e + NS * CHUNK - 1) // (NS * CHUNK)
    cpt_sum = ((cpt_sum + 7) // 8) * 8
    cpt0 = min(max(int(round(cpt_sum * CORE0_FRAC / 8.0)) * 8, 0), cpt_sum)
    cpt1 = cpt_sum - cpt0
    rows = NS * cpt_sum + SR
    e_pad = rows * CHUNK

    row = edge_index[0].astype(jnp.int32)
    col = edge_index[1].astype(jnp.int32)
    pk = jnp.left_shift(row, 16) | col
    pk_p = jnp.concatenate(
        [pk, jnp.full((e_pad - e,), (n_pad - 1) << 16, jnp.int32)])
    pk2d = pk_p.reshape(rows, CHUNK)
    zeros = jnp.zeros((CHUNK, D), jnp.float32)

    b1r = b1.reshape(1, D)
    b2r = b2.reshape(1, D)

    h = _mlp1(x, W1, b1r, W2, b2r, blk=2000)
    parts = _make_sc_scatter(n_pad, cpt0, cpt1)(h, pk2d, zeros)
    return _mlp2(x, parts, W1, b1r, W2, b2r, blk=2000)
